# Initial kernel scaffold; baseline (speedup 1.0000x reference)
import functools

import jax
import jax.numpy as jnp
from jax import lax
from jax.experimental import pallas as pl
from jax.experimental.pallas import tpu as pltpu
from jax.experimental.pallas import tpu_sc as plsc

N = 10000
E = 320000
D_IN = 128
D_H = 256
D_OUT = 128

N_PAD = 10240
NC, NS = 2, 16
NW = NC * NS
EPW = E // NW
CHUNK = 80
NCHUNK = EPW // CHUNK
GPC = CHUNK // 16
ROWS_PT = N_PAD // NS

_MESH = plsc.VectorSubcoreMesh(core_axis_name="c", subcore_axis_name="s")
_F32 = jnp.float32


def _zero_vec(ref, n):
    def body(i, c):
        ref[pl.ds(i * 16, 16)] = jnp.zeros((16,), _F32)
        return c
    lax.fori_loop(0, n // 16, body, 0)


def _deg_body(row_hbm, ew_hbm, out_hbm, row_v, ew_v, deg_v):
    cid = lax.axis_index("c")
    sid = lax.axis_index("s")
    wid = sid * NC + cid
    pltpu.sync_copy(row_hbm.at[wid], row_v)
    pltpu.sync_copy(ew_hbm.at[wid], ew_v)
    _zero_vec(deg_v, N_PAD)

    def body(i, c):
        r = row_v[pl.ds(i * 16, 16)]
        w = ew_v[pl.ds(i * 16, 16)]
        plsc.addupdate_scatter(deg_v, [r], w)
        return c
    lax.fori_loop(0, EPW // 16, body, 0)
    pltpu.sync_copy(deg_v, out_hbm.at[wid])


_deg_call = pl.kernel(
    _deg_body,
    out_type=jax.ShapeDtypeStruct((NW, N_PAD), _F32),
    mesh=_MESH,
    scratch_types=[
        pltpu.VMEM((EPW,), jnp.int32),
        pltpu.VMEM((EPW,), _F32),
        pltpu.VMEM((N_PAD,), _F32),
    ],
)


def _dis_body(parts_ref, dis_ref):
    deg = jnp.sum(parts_ref[...], axis=0, keepdims=True)
    dis_ref[...] = jnp.where(deg > 0.0, lax.rsqrt(deg), 0.0)


def _dis_call(parts):
    return pl.pallas_call(
        _dis_body,
        out_shape=jax.ShapeDtypeStruct((1, N_PAD), _F32),
    )(parts)


def _make_prop(with_s: bool):
    def body(h_hbm, row_hbm, col_hbm, ew_hbm, dis_hbm, *rest):
        if with_s:
            out_hbm, s_hbm, row_v, col_v, ew_v, dis_v, gbuf, s_acc, acc_sh, sem = rest
        else:
            out_hbm, row_v, col_v, ew_v, dis_v, gbuf, acc_sh, sem = rest
        cid = lax.axis_index("c")
        sid = lax.axis_index("s")
        wid = sid * NC + cid
        pltpu.sync_copy(row_hbm.at[wid], row_v)
        pltpu.sync_copy(col_hbm.at[wid], col_v)
        pltpu.sync_copy(ew_hbm.at[wid], ew_v)
        pltpu.sync_copy(dis_hbm, dis_v)
        if with_s:
            _zero_vec(s_acc, N_PAD)

        def zrow(r, c):
            for f in range(8):
                gbuf[r, pl.ds(f * 16, 16)] = jnp.zeros((16,), _F32)
            return c
        lax.fori_loop(0, CHUNK, zrow, 0)

        def zsh(j, c):
            pltpu.sync_copy(gbuf, acc_sh.at[pl.ds(sid * ROWS_PT + j * CHUNK, CHUNK)])
            return c
        lax.fori_loop(0, ROWS_PT // CHUNK, zsh, 0)
        plsc.subcore_barrier()

        def chunk_body(ci, c):
            pltpu.async_copy(h_hbm.at[col_v.at[ci]], gbuf, sem).wait()

            def group_body(g, cc):
                base = g * 16
                r16 = row_v[ci, pl.ds(base, 16)]
                c16 = col_v[ci, pl.ds(base, 16)]
                w16 = ew_v[ci, pl.ds(base, 16)]
                nrm = plsc.load_gather(dis_v, [r16]) * w16 * plsc.load_gather(dis_v, [c16])
                if with_s:
                    plsc.addupdate_scatter(s_acc, [r16], nrm)
                for k in range(16):
                    nb = jnp.take(nrm, jnp.full((16,), k, jnp.int32),
                                  mode=lax.GatherScatterMode.PROMISE_IN_BOUNDS)
                    for f in range(8):
                        sl = pl.ds(f * 16, 16)
                        gbuf[base + k, sl] = gbuf[base + k, sl] * nb
                return cc
            lax.fori_loop(0, GPC, group_body, 0)
            pltpu.sync_copy(gbuf, acc_sh.at[row_v.at[ci]], add=True)
            return c
        lax.fori_loop(0, NCHUNK, chunk_body, 0)
        plsc.subcore_barrier()
        pltpu.sync_copy(acc_sh.at[pl.ds(sid * ROWS_PT, ROWS_PT)],
                        out_hbm.at[cid, pl.ds(sid * ROWS_PT, ROWS_PT)])
        if with_s:
            pltpu.sync_copy(s_acc, s_hbm.at[wid])

    out_types = [jax.ShapeDtypeStruct((NC, N_PAD, 128), _F32)]
    scratch = [
        pltpu.VMEM((NCHUNK, CHUNK), jnp.int32),
        pltpu.VMEM((NCHUNK, CHUNK), jnp.int32),
        pltpu.VMEM((NCHUNK, CHUNK), _F32),
        pltpu.VMEM((N_PAD,), _F32),
        pltpu.VMEM((CHUNK, 128), _F32),
    ]
    if with_s:
        out_types.append(jax.ShapeDtypeStruct((NW, N_PAD), _F32))
        scratch.append(pltpu.VMEM((N_PAD,), _F32))
    scratch.append(pltpu.VMEM_SHARED((N_PAD, 128), _F32))
    scratch.append(pltpu.SemaphoreType.DMA)
    return pl.kernel(
        body,
        out_type=tuple(out_types) if with_s else out_types[0],
        mesh=_MESH,
        scratch_types=scratch,
    )


_prop_s_call = _make_prop(True)
_prop_call = _make_prop(False)


def _mid_body(y_ref, s_ref, w1t_ref, b1_ref, g1_ref, bb1_ref,
              w2t_ref, b2_ref, h2_ref):
    y = y_ref[0, :, :] + y_ref[1, :, :]
    s = jnp.sum(s_ref[...], axis=0)
    h = lax.dot(y, w1t_ref[...], precision=lax.Precision.HIGHEST,
                preferred_element_type=_F32)
    h = h + s[:, None] * b1_ref[...]
    mu = jnp.mean(h, axis=1, keepdims=True)
    var = jnp.mean((h - mu) ** 2, axis=1, keepdims=True)
    a = (h - mu) * lax.rsqrt(var + 1e-5) * g1_ref[...] + bb1_ref[...]
    a = jnp.maximum(a, 0.0)
    h2_ref[...] = lax.dot(a, w2t_ref[...], precision=lax.Precision.HIGHEST,
                          preferred_element_type=_F32) + b2_ref[...]


def _mid_call(y_parts, s_parts, w1t, b1, g1, bb1, w2t, b2):
    grid = N_PAD // 128
    return pl.pallas_call(
        _mid_body,
        grid=(grid,),
        in_specs=[
            pl.BlockSpec((NC, 128, 128), lambda i: (0, i, 0)),
            pl.BlockSpec((NW, 128), lambda i: (0, i)),
            pl.BlockSpec((D_IN, D_H), lambda i: (0, 0)),
            pl.BlockSpec((1, D_H), lambda i: (0, 0)),
            pl.BlockSpec((1, D_H), lambda i: (0, 0)),
            pl.BlockSpec((1, D_H), lambda i: (0, 0)),
            pl.BlockSpec((D_H, D_OUT), lambda i: (0, 0)),
            pl.BlockSpec((1, D_OUT), lambda i: (0, 0)),
        ],
        out_specs=pl.BlockSpec((128, D_OUT), lambda i: (i, 0)),
        out_shape=jax.ShapeDtypeStruct((N_PAD, D_OUT), _F32),
    )(y_parts, s_parts, w1t, b1, g1, bb1, w2t, b2)


def _fin_body(p_ref, g2_ref, bb2_ref, o_ref):
    o = p_ref[0, :, :] + p_ref[1, :, :]
    mu = jnp.mean(o, axis=1, keepdims=True)
    var = jnp.mean((o - mu) ** 2, axis=1, keepdims=True)
    o_ref[...] = (o - mu) * lax.rsqrt(var + 1e-5) * g2_ref[...] + bb2_ref[...]


def _fin_call(p_parts, g2, bb2):
    grid = N_PAD // 128
    return pl.pallas_call(
        _fin_body,
        grid=(grid,),
        in_specs=[
            pl.BlockSpec((NC, 128, D_OUT), lambda i: (0, i, 0)),
            pl.BlockSpec((1, D_OUT), lambda i: (0, 0)),
            pl.BlockSpec((1, D_OUT), lambda i: (0, 0)),
        ],
        out_specs=pl.BlockSpec((128, D_OUT), lambda i: (i, 0)),
        out_shape=jax.ShapeDtypeStruct((N_PAD, D_OUT), _F32),
    )(p_parts, g2, bb2)


def kernel(x, edge_index, edge_weight, W1, b1, W2, b2, ln1_g, ln1_b, ln2_g, ln2_b):
    row = edge_index[0].astype(jnp.int32)
    col = edge_index[1].astype(jnp.int32)
    row3 = row.reshape(NW, NCHUNK, CHUNK)
    col3 = col.reshape(NW, NCHUNK, CHUNK)
    ew3 = edge_weight.reshape(NW, NCHUNK, CHUNK)

    x_pad = jnp.zeros((N_PAD, D_IN), _F32).at[:N].set(x)

    deg_parts = _deg_call(row.reshape(NW, EPW), edge_weight.reshape(NW, EPW))
    dis = _dis_call(deg_parts).reshape(N_PAD)

    y1_parts, s_parts = _prop_s_call(x_pad, row3, col3, ew3, dis)
    h2 = _mid_call(y1_parts, s_parts, W1.T, b1.reshape(1, -1),
                   ln1_g.reshape(1, -1), ln1_b.reshape(1, -1),
                   W2.T, b2.reshape(1, -1))
    y2_parts = _prop_call(h2, row3, col3, ew3, dis)
    out = _fin_call(y2_parts, ln2_g.reshape(1, -1), ln2_b.reshape(1, -1))
    return out[:N]


# speedup vs baseline: 17.6666x; 17.6666x over previous
import functools

import jax
import jax.numpy as jnp
from jax import lax
from jax.experimental import pallas as pl
from jax.experimental.pallas import tpu as pltpu
from jax.experimental.pallas import tpu_sc as plsc

N = 10000
E = 320000
D_IN = 128
D_H = 256
D_OUT = 128

N_PAD = 10240
NC, NS = 2, 16
NW = NC * NS
EPW = E // NW
CHUNK = 80
NCHUNK = EPW // CHUNK
GPC = CHUNK // 16
NBLK = 5
CPB = NCHUNK // NBLK
ROWS_PT = N_PAD // NS

_F32 = jnp.float32
_SC_PARAMS = pltpu.CompilerParams(use_tc_tiling_on_sc=False,
                                 needs_layout_passes=False)


@functools.cache
def _mesh():
    return plsc.VectorSubcoreMesh(core_axis_name="c", subcore_axis_name="s",
                                  num_cores=NC, num_subcores=NS)


def _zero_vec(ref, n):
    def body(i, c):
        ref[pl.ds(i * 16, 16)] = jnp.zeros((16,), _F32)
        return c
    lax.fori_loop(0, n // 16, body, 0)


DIS_R = N_PAD // 128


def _rsqrt_nr(d):
    bits = plsc.bitcast(d, jnp.int32)
    y = plsc.bitcast(jnp.full((16,), 0x5F3759DF, jnp.int32)
                     - jnp.right_shift(bits, 1), _F32)
    for _ in range(3):
        y = y * (1.5 - 0.5 * d * y * y)
    return jnp.where(d > 0.0, y, 0.0)


def _split_idx(i16):
    return jnp.right_shift(i16, 7), jnp.bitwise_and(i16, 127)


def _make_prop(first: bool):
    def body(h_hbm, row_hbm, col_hbm, ew_hbm, *rest):
        if first:
            (out_hbm, s_hbm, dis_out_hbm, row_v, col_v, ew_v, dis_v,
             gbuf_a, gbuf_b, s_acc, tbuf, dbuf, acc_sh, sem) = rest
        else:
            (dis_hbm, out_hbm, row_v, col_v, ew_v, dis_v,
             gbuf_a, gbuf_b, acc_sh, sem) = rest
        cid = lax.axis_index("c")
        sid = lax.axis_index("s")
        wid = sid * NC + cid

        if first:
            def zdis(r, c):
                for f in range(8):
                    dis_v[r, pl.ds(f * 16, 16)] = jnp.zeros((16,), _F32)
                return c
            lax.fori_loop(0, DIS_R, zdis, 0)

            for wo in range(2):
                w = sid * 2 + wo

                def dblk(b, c):
                    pltpu.sync_copy(row_hbm.at[w, pl.ds(b * CPB, CPB)], row_v)
                    pltpu.sync_copy(ew_hbm.at[w, pl.ds(b * CPB, CPB)], ew_v)

                    def dchunk(ci, cc):
                        def dgrp(g, ccc):
                            base = g * 16
                            r16 = row_v[ci, pl.ds(base, 16)]
                            w16 = ew_v[ci, pl.ds(base, 16)]
                            hi, lo = _split_idx(r16)
                            plsc.addupdate_scatter(dis_v, [hi, lo], w16)
                            return ccc
                        lax.fori_loop(0, GPC, dgrp, 0)
                        return cc
                    lax.fori_loop(0, CPB, dchunk, 0)
                    return c
                lax.fori_loop(0, NBLK, dblk, 0)

            pltpu.sync_copy(dis_v, acc_sh.at[pl.ds(sid * DIS_R, DIS_R)])
            plsc.subcore_barrier()
            for r in range(5):
                for f in range(8):
                    dbuf[r, pl.ds(f * 16, 16)] = jnp.zeros((16,), _F32)

            def redp(p, c):
                pltpu.sync_copy(acc_sh.at[pl.ds(p * DIS_R + sid * 5, 5)], tbuf)
                for r in range(5):
                    for f in range(8):
                        sl = pl.ds(f * 16, 16)
                        dbuf[r, sl] = dbuf[r, sl] + tbuf[r, sl]
                return c
            lax.fori_loop(0, NS, redp, 0)
            for r in range(5):
                for f in range(8):
                    sl = pl.ds(f * 16, 16)
                    dbuf[r, sl] = _rsqrt_nr(dbuf[r, sl])

            pltpu.sync_copy(dbuf, acc_sh.at[pl.ds(NS * DIS_R + sid * 5, 5)])
            plsc.subcore_barrier()
            pltpu.sync_copy(acc_sh.at[pl.ds(NS * DIS_R, DIS_R)], dis_v)

            @pl.when(jnp.logical_and(sid == 0, cid == 0))
            def _():
                pltpu.sync_copy(dis_v, dis_out_hbm)
            plsc.subcore_barrier()
            _zero_vec(s_acc, N_PAD)
        else:
            pltpu.sync_copy(dis_hbm, dis_v)

        def zrow(r, c):
            for f in range(8):
                gbuf_a[r, pl.ds(f * 16, 16)] = jnp.zeros((16,), _F32)
            return c
        lax.fori_loop(0, CHUNK, zrow, 0)

        def zsh(j, c):
            pltpu.sync_copy(gbuf_a,
                            acc_sh.at[pl.ds(sid * ROWS_PT + j * CHUNK, CHUNK)])
            return c
        lax.fori_loop(0, ROWS_PT // CHUNK, zsh, 0)
        plsc.subcore_barrier()

        def compute_scatter(ci, gbuf):
            def group_body(g, cc):
                base = g * 16
                r16 = row_v[ci, pl.ds(base, 16)]
                c16 = col_v[ci, pl.ds(base, 16)]
                w16 = ew_v[ci, pl.ds(base, 16)]
                rhi, rlo = _split_idx(r16)
                chi, clo = _split_idx(c16)
                nrm = (plsc.load_gather(dis_v, [rhi, rlo]) * w16
                       * plsc.load_gather(dis_v, [chi, clo]))
                if first:
                    plsc.addupdate_scatter(s_acc, [r16], nrm)
                for k in range(16):
                    nb = jnp.broadcast_to(nrm[k], (16,))
                    for f in range(8):
                        sl = pl.ds(f * 16, 16)
                        gbuf[base + k, sl] = gbuf[base + k, sl] * nb
                return cc
            lax.fori_loop(0, GPC, group_body, 0)
            pltpu.sync_copy(gbuf, acc_sh.at[row_v.at[ci]], add=True)

        def issue(ci, gbuf):
            pltpu.async_copy(h_hbm.at[col_v.at[ci]], gbuf, sem)

        def wait(ci, gbuf):
            pltpu.make_async_copy(h_hbm.at[col_v.at[ci]], gbuf, sem).wait()

        def blk_body(b, c0):
            pltpu.sync_copy(row_hbm.at[wid, pl.ds(b * CPB, CPB)], row_v)
            pltpu.sync_copy(col_hbm.at[wid, pl.ds(b * CPB, CPB)], col_v)
            pltpu.sync_copy(ew_hbm.at[wid, pl.ds(b * CPB, CPB)], ew_v)
            issue(0, gbuf_a)

            def pair_body(jj, c):
                a = 2 * jj
                wait(a, gbuf_a)
                issue(a + 1, gbuf_b)
                compute_scatter(a, gbuf_a)
                wait(a + 1, gbuf_b)
                issue(a + 2, gbuf_a)
                compute_scatter(a + 1, gbuf_b)
                return c
            lax.fori_loop(0, CPB // 2, pair_body, 0)
            wait(CPB - 1, gbuf_a)
            compute_scatter(CPB - 1, gbuf_a)
            return c0
        lax.fori_loop(0, NBLK, blk_body, 0)
        plsc.subcore_barrier()
        pltpu.sync_copy(acc_sh.at[pl.ds(sid * ROWS_PT, ROWS_PT)],
                        out_hbm.at[cid, pl.ds(sid * ROWS_PT, ROWS_PT)])
        if first:
            pltpu.sync_copy(s_acc, s_hbm.at[wid])

    scratch = [
        pltpu.VMEM((CPB, CHUNK), jnp.int32),
        pltpu.VMEM((CPB, CHUNK), jnp.int32),
        pltpu.VMEM((CPB, CHUNK), _F32),
        pltpu.VMEM((DIS_R, 128), _F32),
        pltpu.VMEM((CHUNK, 128), _F32),
        pltpu.VMEM((CHUNK, 128), _F32),
    ]
    if first:
        out_types = (jax.ShapeDtypeStruct((NC, N_PAD, 128), _F32),
                     jax.ShapeDtypeStruct((NW, N_PAD), _F32),
                     jax.ShapeDtypeStruct((DIS_R, 128), _F32))
        scratch += [
            pltpu.VMEM((N_PAD,), _F32),
            pltpu.VMEM((5, 128), _F32),
            pltpu.VMEM((5, 128), _F32),
        ]
    else:
        out_types = jax.ShapeDtypeStruct((NC, N_PAD, 128), _F32)
    scratch.append(pltpu.VMEM_SHARED((N_PAD, 128), _F32))
    scratch.append(pltpu.SemaphoreType.DMA)
    return pl.kernel(
        body,
        out_type=out_types,
        mesh=_mesh(),
        compiler_params=_SC_PARAMS,
        scratch_types=scratch,
    )


_make_prop = functools.cache(_make_prop)


def _mid_body(y_ref, s_ref, w1t_ref, b1_ref, g1_ref, bb1_ref,
              w2t_ref, b2_ref, h2_ref):
    y = y_ref[0, :, :] + y_ref[1, :, :]
    s = jnp.sum(s_ref[...], axis=0)
    h = lax.dot(y, w1t_ref[...], precision=lax.Precision.DEFAULT,
                preferred_element_type=_F32)
    h = h + s[:, None] * b1_ref[...]
    mu = jnp.mean(h, axis=1, keepdims=True)
    var = jnp.mean((h - mu) ** 2, axis=1, keepdims=True)
    a = (h - mu) * lax.rsqrt(var + 1e-5) * g1_ref[...] + bb1_ref[...]
    a = jnp.maximum(a, 0.0)
    h2_ref[...] = lax.dot(a, w2t_ref[...], precision=lax.Precision.DEFAULT,
                          preferred_element_type=_F32) + b2_ref[...]


def _mid_call(y_parts, s_parts, w1t, b1, g1, bb1, w2t, b2):
    grid = N_PAD // 128
    return pl.pallas_call(
        _mid_body,
        grid=(grid,),
        in_specs=[
            pl.BlockSpec((NC, 128, 128), lambda i: (0, i, 0)),
            pl.BlockSpec((NW, 128), lambda i: (0, i)),
            pl.BlockSpec((D_IN, D_H), lambda i: (0, 0)),
            pl.BlockSpec((1, D_H), lambda i: (0, 0)),
            pl.BlockSpec((1, D_H), lambda i: (0, 0)),
            pl.BlockSpec((1, D_H), lambda i: (0, 0)),
            pl.BlockSpec((D_H, D_OUT), lambda i: (0, 0)),
            pl.BlockSpec((1, D_OUT), lambda i: (0, 0)),
        ],
        out_specs=pl.BlockSpec((128, D_OUT), lambda i: (i, 0)),
        out_shape=jax.ShapeDtypeStruct((N_PAD, D_OUT), _F32),
    )(y_parts, s_parts, w1t, b1, g1, bb1, w2t, b2)


def _fin_body(p_ref, g2_ref, bb2_ref, o_ref):
    o = p_ref[0, :, :] + p_ref[1, :, :]
    mu = jnp.mean(o, axis=1, keepdims=True)
    var = jnp.mean((o - mu) ** 2, axis=1, keepdims=True)
    o_ref[...] = (o - mu) * lax.rsqrt(var + 1e-5) * g2_ref[...] + bb2_ref[...]


def _fin_call(p_parts, g2, bb2):
    grid = N_PAD // 128
    return pl.pallas_call(
        _fin_body,
        grid=(grid,),
        in_specs=[
            pl.BlockSpec((NC, 128, D_OUT), lambda i: (0, i, 0)),
            pl.BlockSpec((1, D_OUT), lambda i: (0, 0)),
            pl.BlockSpec((1, D_OUT), lambda i: (0, 0)),
        ],
        out_specs=pl.BlockSpec((128, D_OUT), lambda i: (i, 0)),
        out_shape=jax.ShapeDtypeStruct((N_PAD, D_OUT), _F32),
    )(p_parts, g2, bb2)


def kernel(x, edge_index, edge_weight, W1, b1, W2, b2, ln1_g, ln1_b, ln2_g, ln2_b):
    row = edge_index[0].astype(jnp.int32)
    col = edge_index[1].astype(jnp.int32)
    row3 = row.reshape(NW, NCHUNK, CHUNK)
    col3 = col.reshape(NW, NCHUNK, CHUNK)
    ew3 = edge_weight.reshape(NW, NCHUNK, CHUNK)

    x_pad = jnp.zeros((N_PAD, D_IN), _F32).at[:N].set(x)

    y1_parts, s_parts, dis = _make_prop(True)(x_pad, row3, col3, ew3)
    h2 = _mid_call(y1_parts, s_parts, W1.T, b1.reshape(1, -1),
                   ln1_g.reshape(1, -1), ln1_b.reshape(1, -1),
                   W2.T, b2.reshape(1, -1))
    y2_parts = _make_prop(False)(h2, row3, col3, ew3, dis)
    out = _fin_call(y2_parts, ln2_g.reshape(1, -1), ln2_b.reshape(1, -1))
    return out[:N]


# speedup vs baseline: 18.0322x; 1.0207x over previous
import functools

import jax
import jax.numpy as jnp
from jax import lax
from jax.experimental import pallas as pl
from jax.experimental.pallas import tpu as pltpu
from jax.experimental.pallas import tpu_sc as plsc

N = 10000
E = 320000
D_IN = 128
D_H = 256
D_OUT = 128

N_PAD = 10240
NC, NS = 2, 16
NW = NC * NS
EPW = E // NW
CHUNK = 80
NCHUNK = EPW // CHUNK
GPC = CHUNK // 16
NBLK = 5
CPB = NCHUNK // NBLK
ROWS_PT = N_PAD // NS

_F32 = jnp.float32
_SC_PARAMS = pltpu.CompilerParams(use_tc_tiling_on_sc=False,
                                 needs_layout_passes=False)


@functools.cache
def _mesh():
    return plsc.VectorSubcoreMesh(core_axis_name="c", subcore_axis_name="s",
                                  num_cores=NC, num_subcores=NS)


def _zero_vec(ref, n):
    def body(i, c):
        ref[pl.ds(i * 16, 16)] = jnp.zeros((16,), _F32)
        return c
    lax.fori_loop(0, n // 16, body, 0)


def _deg_body(row_hbm, ew_hbm, out_hbm, row_v, ew_v, deg_v):
    cid = lax.axis_index("c")
    sid = lax.axis_index("s")
    wid = sid * NC + cid
    pltpu.sync_copy(row_hbm.at[wid], row_v)
    pltpu.sync_copy(ew_hbm.at[wid], ew_v)
    _zero_vec(deg_v, N_PAD)

    def body(i, c):
        r = row_v[pl.ds(i * 16, 16)]
        w = ew_v[pl.ds(i * 16, 16)]
        plsc.addupdate_scatter(deg_v, [r], w)
        return c
    lax.fori_loop(0, EPW // 16, body, 0)
    pltpu.sync_copy(deg_v, out_hbm.at[wid])


@functools.cache
def _deg_call():
    return pl.kernel(
        _deg_body,
        out_type=jax.ShapeDtypeStruct((NW, N_PAD), _F32),
        mesh=_mesh(),
        compiler_params=_SC_PARAMS,
        scratch_types=[
            pltpu.VMEM((EPW,), jnp.int32),
            pltpu.VMEM((EPW,), _F32),
            pltpu.VMEM((N_PAD,), _F32),
        ],
    )


def _dis_body(parts_ref, dis_ref):
    deg = jnp.sum(parts_ref[...], axis=0, keepdims=True)
    dis_ref[...] = jnp.where(deg > 0.0, lax.rsqrt(deg), 0.0)


def _dis_call(parts):
    return pl.pallas_call(
        _dis_body,
        out_shape=jax.ShapeDtypeStruct((1, N_PAD), _F32),
    )(parts)


DIS_R = N_PAD // 128


def _rsqrt_nr(d):
    bits = plsc.bitcast(d, jnp.int32)
    y = plsc.bitcast(jnp.full((16,), 0x5F3759DF, jnp.int32)
                     - jnp.right_shift(bits, 1), _F32)
    for _ in range(3):
        y = y * (1.5 - 0.5 * d * y * y)
    return jnp.where(d > 0.0, y, 0.0)


def _split_idx(i16):
    return jnp.right_shift(i16, 7), jnp.bitwise_and(i16, 127)


def _make_prop(first: bool):
    def body(h_hbm, row_hbm, col_hbm, ew_hbm, dis_hbm, *rest):
        if first:
            (out_hbm, s_hbm, row_v, col_v, ew_v, dis_v,
             gbuf_a, gbuf_b, s_acc, acc_sh, sem) = rest
        else:
            (out_hbm, row_v, col_v, ew_v, dis_v,
             gbuf_a, gbuf_b, acc_sh, sem) = rest
        cid = lax.axis_index("c")
        sid = lax.axis_index("s")
        wid = sid * NC + cid
        pltpu.sync_copy(dis_hbm, dis_v)

        if first:
            _zero_vec(s_acc, N_PAD)

        def zrow(r, c):
            for f in range(8):
                gbuf_a[r, pl.ds(f * 16, 16)] = jnp.zeros((16,), _F32)
            return c
        lax.fori_loop(0, CHUNK, zrow, 0)

        def zsh(j, c):
            pltpu.sync_copy(gbuf_a,
                            acc_sh.at[pl.ds(sid * ROWS_PT + j * CHUNK, CHUNK)])
            return c
        lax.fori_loop(0, ROWS_PT // CHUNK, zsh, 0)
        plsc.subcore_barrier()

        def compute_scatter(ci, gbuf):
            def group_body(g, cc):
                base = g * 16
                r16 = row_v[ci, pl.ds(base, 16)]
                c16 = col_v[ci, pl.ds(base, 16)]
                w16 = ew_v[ci, pl.ds(base, 16)]
                rhi, rlo = _split_idx(r16)
                chi, clo = _split_idx(c16)
                nrm = (plsc.load_gather(dis_v, [rhi, rlo]) * w16
                       * plsc.load_gather(dis_v, [chi, clo]))
                if first:
                    plsc.addupdate_scatter(s_acc, [r16], nrm)
                for k in range(16):
                    nb = jnp.broadcast_to(nrm[k], (16,))
                    for f in range(8):
                        sl = pl.ds(f * 16, 16)
                        gbuf[base + k, sl] = gbuf[base + k, sl] * nb
                return cc
            lax.fori_loop(0, GPC, group_body, 0)
            pltpu.sync_copy(gbuf, acc_sh.at[row_v.at[ci]], add=True)

        def issue(ci, gbuf):
            pltpu.async_copy(h_hbm.at[col_v.at[ci]], gbuf, sem)

        def wait(ci, gbuf):
            pltpu.make_async_copy(h_hbm.at[col_v.at[ci]], gbuf, sem).wait()

        def blk_body(b, c0):
            pltpu.sync_copy(row_hbm.at[wid, pl.ds(b * CPB, CPB)], row_v)
            pltpu.sync_copy(col_hbm.at[wid, pl.ds(b * CPB, CPB)], col_v)
            pltpu.sync_copy(ew_hbm.at[wid, pl.ds(b * CPB, CPB)], ew_v)
            issue(0, gbuf_a)

            def pair_body(jj, c):
                a = 2 * jj
                wait(a, gbuf_a)
                issue(a + 1, gbuf_b)
                compute_scatter(a, gbuf_a)
                wait(a + 1, gbuf_b)
                issue(a + 2, gbuf_a)
                compute_scatter(a + 1, gbuf_b)
                return c
            lax.fori_loop(0, CPB // 2, pair_body, 0)
            wait(CPB - 1, gbuf_a)
            compute_scatter(CPB - 1, gbuf_a)
            return c0
        lax.fori_loop(0, NBLK, blk_body, 0)
        plsc.subcore_barrier()
        pltpu.sync_copy(acc_sh.at[pl.ds(sid * ROWS_PT, ROWS_PT)],
                        out_hbm.at[cid, pl.ds(sid * ROWS_PT, ROWS_PT)])
        if first:
            pltpu.sync_copy(s_acc, s_hbm.at[wid])

    scratch = [
        pltpu.VMEM((CPB, CHUNK), jnp.int32),
        pltpu.VMEM((CPB, CHUNK), jnp.int32),
        pltpu.VMEM((CPB, CHUNK), _F32),
        pltpu.VMEM((DIS_R, 128), _F32),
        pltpu.VMEM((CHUNK, 128), _F32),
        pltpu.VMEM((CHUNK, 128), _F32),
    ]
    if first:
        out_types = (jax.ShapeDtypeStruct((NC, N_PAD, 128), _F32),
                     jax.ShapeDtypeStruct((NW, N_PAD), _F32))
        scratch.append(pltpu.VMEM((N_PAD,), _F32))
    else:
        out_types = jax.ShapeDtypeStruct((NC, N_PAD, 128), _F32)
    scratch.append(pltpu.VMEM_SHARED((N_PAD, 128), _F32))
    scratch.append(pltpu.SemaphoreType.DMA)
    return pl.kernel(
        body,
        out_type=out_types,
        mesh=_mesh(),
        compiler_params=_SC_PARAMS,
        scratch_types=scratch,
    )


_make_prop = functools.cache(_make_prop)


def _mid_body(y_ref, s_ref, w1t_ref, b1_ref, g1_ref, bb1_ref,
              w2t_ref, b2_ref, h2_ref):
    y = y_ref[0, :, :] + y_ref[1, :, :]
    s = jnp.sum(s_ref[...], axis=0)
    h = lax.dot(y, w1t_ref[...], precision=lax.Precision.DEFAULT,
                preferred_element_type=_F32)
    h = h + s[:, None] * b1_ref[...]
    mu = jnp.mean(h, axis=1, keepdims=True)
    var = jnp.mean((h - mu) ** 2, axis=1, keepdims=True)
    a = (h - mu) * lax.rsqrt(var + 1e-5) * g1_ref[...] + bb1_ref[...]
    a = jnp.maximum(a, 0.0)
    h2_ref[...] = lax.dot(a, w2t_ref[...], precision=lax.Precision.DEFAULT,
                          preferred_element_type=_F32) + b2_ref[...]


def _mid_call(y_parts, s_parts, w1t, b1, g1, bb1, w2t, b2):
    grid = N_PAD // 128
    return pl.pallas_call(
        _mid_body,
        grid=(grid,),
        in_specs=[
            pl.BlockSpec((NC, 128, 128), lambda i: (0, i, 0)),
            pl.BlockSpec((NW, 128), lambda i: (0, i)),
            pl.BlockSpec((D_IN, D_H), lambda i: (0, 0)),
            pl.BlockSpec((1, D_H), lambda i: (0, 0)),
            pl.BlockSpec((1, D_H), lambda i: (0, 0)),
            pl.BlockSpec((1, D_H), lambda i: (0, 0)),
            pl.BlockSpec((D_H, D_OUT), lambda i: (0, 0)),
            pl.BlockSpec((1, D_OUT), lambda i: (0, 0)),
        ],
        out_specs=pl.BlockSpec((128, D_OUT), lambda i: (i, 0)),
        out_shape=jax.ShapeDtypeStruct((N_PAD, D_OUT), _F32),
    )(y_parts, s_parts, w1t, b1, g1, bb1, w2t, b2)


def _fin_body(p_ref, g2_ref, bb2_ref, o_ref):
    o = p_ref[0, :, :] + p_ref[1, :, :]
    mu = jnp.mean(o, axis=1, keepdims=True)
    var = jnp.mean((o - mu) ** 2, axis=1, keepdims=True)
    o_ref[...] = (o - mu) * lax.rsqrt(var + 1e-5) * g2_ref[...] + bb2_ref[...]


def _fin_call(p_parts, g2, bb2):
    grid = N_PAD // 128
    return pl.pallas_call(
        _fin_body,
        grid=(grid,),
        in_specs=[
            pl.BlockSpec((NC, 128, D_OUT), lambda i: (0, i, 0)),
            pl.BlockSpec((1, D_OUT), lambda i: (0, 0)),
            pl.BlockSpec((1, D_OUT), lambda i: (0, 0)),
        ],
        out_specs=pl.BlockSpec((128, D_OUT), lambda i: (i, 0)),
        out_shape=jax.ShapeDtypeStruct((N_PAD, D_OUT), _F32),
    )(p_parts, g2, bb2)


def kernel(x, edge_index, edge_weight, W1, b1, W2, b2, ln1_g, ln1_b, ln2_g, ln2_b):
    row = edge_index[0].astype(jnp.int32)
    col = edge_index[1].astype(jnp.int32)
    row3 = row.reshape(NW, NCHUNK, CHUNK)
    col3 = col.reshape(NW, NCHUNK, CHUNK)
    ew3 = edge_weight.reshape(NW, NCHUNK, CHUNK)

    x_pad = jnp.zeros((N_PAD, D_IN), _F32).at[:N].set(x)

    deg_parts = _deg_call()(row.reshape(NW, EPW), edge_weight.reshape(NW, EPW))
    dis = _dis_call(deg_parts).reshape(DIS_R, 128)

    y1_parts, s_parts = _make_prop(True)(x_pad, row3, col3, ew3, dis)
    h2 = _mid_call(y1_parts, s_parts, W1.T, b1.reshape(1, -1),
                   ln1_g.reshape(1, -1), ln1_b.reshape(1, -1),
                   W2.T, b2.reshape(1, -1))
    y2_parts = _make_prop(False)(h2, row3, col3, ew3, dis)
    out = _fin_call(y2_parts, ln2_g.reshape(1, -1), ln2_b.reshape(1, -1))
    return out[:N]
